# baseline (speedup 1.0000x reference)
import functools

import jax
import jax.numpy as jnp
from jax import lax
from jax.experimental import pallas as pl
from jax.experimental.pallas import tpu as pltpu
from jax.experimental.pallas import tpu_sc as plsc

D = 64
K = 50
LANES = 16
CHUNK = 8
NBUF = 4
CHUNK_IDX = CHUNK * K


def _make_agg(B: int):
    info = plsc.get_sparse_core_info()
    NC, NS = info.num_cores, info.num_subcores
    NW = NC * NS
    assert B % (NW * CHUNK * NBUF) == 0
    b_per_w = B // NW
    n_chunks = b_per_w // CHUNK

    mesh = plsc.VectorSubcoreMesh(core_axis_name="c", subcore_axis_name="s")

    @functools.partial(
        pl.kernel,
        mesh=mesh,
        compiler_params=pltpu.CompilerParams(use_tc_tiling_on_sc=False),
        out_type=jax.ShapeDtypeStruct((B, D), jnp.float32),
        scratch_types=[
            pltpu.VMEM((NBUF, CHUNK_IDX), jnp.int32),
            pltpu.VMEM((b_per_w,), jnp.int32),
            pltpu.VMEM((NBUF, CHUNK_IDX, D), jnp.float32),
            pltpu.VMEM((NBUF, CHUNK, D), jnp.float32),
            pltpu.VMEM((NBUF, CHUNK, D), jnp.float32),
            [pltpu.SemaphoreType.DMA] * NBUF,
            [pltpu.SemaphoreType.DMA] * NBUF,
            [pltpu.SemaphoreType.DMA] * NBUF,
        ],
    )
    def agg(item_idx_hbm, nbr_idx_hbm, item_tab_hbm, ent_tab_hbm, out_hbm,
            nidx_v, iidx_v, nrows_v, irows_v, out_v, gsem, isem, osem):
        wid = lax.axis_index("s") * NC + lax.axis_index("c")
        row0 = wid * b_per_w
        pltpu.sync_copy(item_idx_hbm.at[pl.ds(row0, b_per_w)], iidx_v)

        def fire_idx(g, s):
            pltpu.async_copy(
                nbr_idx_hbm.at[pl.ds((row0 + g * CHUNK) * K, CHUNK_IDX)],
                nidx_v.at[s], isem[s])

        def wait_idx(s):
            pltpu.make_async_copy(
                nbr_idx_hbm.at[pl.ds(0, CHUNK_IDX)], nidx_v.at[s],
                isem[s]).wait()

        def fire_gather(g, s):
            pltpu.async_copy(ent_tab_hbm.at[nidx_v.at[s]], nrows_v.at[s],
                             gsem[s])
            pltpu.async_copy(
                item_tab_hbm.at[iidx_v.at[pl.ds(g * CHUNK, CHUNK)]],
                irows_v.at[s], gsem[s])

        def wait_gather(b):
            pltpu.make_async_copy(
                ent_tab_hbm.at[pl.ds(0, CHUNK_IDX)], nrows_v.at[b],
                gsem[b]).wait()
            pltpu.make_async_copy(
                item_tab_hbm.at[pl.ds(0, CHUNK)], irows_v.at[b],
                gsem[b]).wait()

        def wait_out(b):
            pltpu.make_async_copy(
                out_v.at[b], out_hbm.at[pl.ds(0, CHUNK), :], osem[b]).wait()

        for c in range(3):
            fire_idx(c, c)
        for c in range(2):
            wait_idx(c)
            fire_gather(c, c)

        @pl.loop(0, n_chunks, step=NBUF)
        def outer(i):
            for b in range(NBUF):
                g = i + b
                wait_gather(b)

                @pl.when(g + 3 < n_chunks)
                def _():
                    fire_idx(g + 3, (b + 3) % NBUF)

                @pl.when(g + 2 < n_chunks)
                def _():
                    wait_idx((b + 2) % NBUF)
                    fire_gather(g + 2, (b + 2) % NBUF)

                @pl.when(g >= NBUF)
                def _():
                    wait_out(b)

                def row_body(r, _):
                    base = r * K
                    for q in range(D // LANES):
                        c = pl.ds(q * LANES, LANES)
                        acc = nrows_v[b, base, c]
                        for k in range(1, K):
                            acc = acc + nrows_v[b, base + k, c]
                        out_v[b, r, c] = irows_v[b, r, c] + acc * (1.0 / K)
                    return 0

                lax.fori_loop(0, CHUNK, row_body, 0, unroll=2)
                pltpu.async_copy(
                    out_v.at[b], out_hbm.at[pl.ds(row0 + g * CHUNK, CHUNK), :],
                    osem[b])

        for b in range(NBUF):
            wait_out(b)

    return agg


TBLK = 8192
TBLK_BITS = 13


def _tr_body(a_ref, b_ref, o_ref):
    o_ref[...] = jnp.concatenate([a_ref[...].T, b_ref[...].T], axis=1)


def _make_tr(R: int):
    npairs = (R + 2 * TBLK - 1) // (2 * TBLK)
    nblocks = (R + TBLK - 1) // TBLK
    return pl.pallas_call(
        _tr_body,
        grid=(npairs,),
        in_specs=[
            pl.BlockSpec((D, TBLK), lambda j: (0, 2 * j)),
            pl.BlockSpec((D, TBLK),
                         lambda j: (0, jnp.minimum(2 * j + 1, nblocks - 1))),
        ],
        out_specs=pl.BlockSpec((TBLK, 2 * D), lambda j: (j, 0)),
        out_shape=jax.ShapeDtypeStruct((npairs * TBLK, 2 * D), jnp.float32),
    )


def _remap(i):
    blk = i >> TBLK_BITS
    return ((((blk >> 1) << TBLK_BITS) + (i & (TBLK - 1))) << 1) + (blk & 1)


def _fc_body(x_ref, w_ref, b_ref, o_ref):
    y = lax.dot_general(x_ref[...], w_ref[...], (((1,), (1,)), ((), ())),
                        preferred_element_type=jnp.float32)
    o_ref[...] = jnp.maximum(y + b_ref[...], 0.0)


def _make_fc(B: int):
    blk = 2048
    return pl.pallas_call(
        _fc_body,
        grid=(B // blk,),
        in_specs=[
            pl.BlockSpec((blk, D), lambda i: (i, 0)),
            pl.BlockSpec((D, D), lambda i: (0, 0)),
            pl.BlockSpec((1, D), lambda i: (0, 0)),
        ],
        out_specs=pl.BlockSpec((blk, D), lambda i: (i, 0)),
        out_shape=jax.ShapeDtypeStruct((B, D), jnp.float32),
    )


def kernel(item_indices, neighbor_indices, item_table, entity_table, fc1_w, fc1_b):
    B = item_indices.shape[0]
    NE = entity_table.shape[0]
    NI = item_table.shape[0]
    ent_t = entity_table.T
    itm_t = item_table.T
    ent_lin = _make_tr(NE)(ent_t, ent_t)
    ent_lin = ent_lin.reshape(2 * ent_lin.shape[0], D)
    itm_lin = _make_tr(NI)(itm_t, itm_t)
    itm_lin = itm_lin.reshape(2 * itm_lin.shape[0], D)
    nb = _remap(neighbor_indices.reshape(-1).astype(jnp.int32))
    ii = _remap(item_indices.astype(jnp.int32))
    agg = _make_agg(B)(ii, nb, itm_lin, ent_lin)
    return _make_fc(B)(agg, fc1_w, fc1_b.reshape(1, D))


# speedup vs baseline: 1.1026x; 1.1026x over previous
import functools

import jax
import jax.numpy as jnp
from jax import lax
from jax.experimental import pallas as pl
from jax.experimental.pallas import tpu as pltpu
from jax.experimental.pallas import tpu_sc as plsc

D = 64
K = 50
LANES = 16
CHUNK = 8
NBUF = 2
CHUNK_IDX = CHUNK * K
IDX_SLICES = [(0, CHUNK_IDX)]


def _make_agg(B: int):
    info = plsc.get_sparse_core_info()
    NC, NS = info.num_cores, info.num_subcores
    NW = NC * NS
    assert B % (NW * CHUNK * NBUF) == 0
    b_per_w = B // NW
    n_chunks = b_per_w // CHUNK

    mesh = plsc.VectorSubcoreMesh(core_axis_name="c", subcore_axis_name="s")

    @functools.partial(
        pl.kernel,
        mesh=mesh,
        compiler_params=pltpu.CompilerParams(use_tc_tiling_on_sc=False),
        out_type=jax.ShapeDtypeStruct((B, 2 * D), jnp.float32),
        scratch_types=[
            pltpu.VMEM((b_per_w * K,), jnp.int32),
            pltpu.VMEM((NBUF, CHUNK_IDX, D), jnp.float32),
            pltpu.VMEM((NBUF, CHUNK, D), jnp.float32),
            [pltpu.SemaphoreType.DMA] * NBUF,
            [pltpu.SemaphoreType.DMA] * NBUF,
        ],
    )
    def agg(nbr_idx_hbm, ent_tab_hbm, out_hbm, nidx_v, nrows_v, out_v,
            gsem, osem):
        wid = lax.axis_index("s") * NC + lax.axis_index("c")
        row0 = wid * b_per_w
        pltpu.sync_copy(nbr_idx_hbm.at[pl.ds(row0 * K, b_per_w * K)], nidx_v)

        def fire(g, b):
            pltpu.async_copy(
                ent_tab_hbm.at[nidx_v.at[pl.ds(g * CHUNK_IDX, CHUNK_IDX)]],
                nrows_v.at[b], gsem[b])

        def wait_gather(b):
            pltpu.make_async_copy(
                ent_tab_hbm.at[pl.ds(0, CHUNK_IDX)], nrows_v.at[b],
                gsem[b]).wait()

        def wait_out(b):
            pltpu.make_async_copy(
                out_v.at[b], out_hbm.at[pl.ds(0, CHUNK), pl.ds(0, D)],
                osem[b]).wait()

        for b in range(NBUF):
            fire(b, b)

        @pl.loop(0, n_chunks, step=NBUF)
        def outer(i):
            for b in range(NBUF):
                g = i + b
                wait_gather(b)

                @pl.when(g >= NBUF)
                def _():
                    wait_out(b)

                def row_body(r, _):
                    base = r * K
                    for q in range(D // LANES):
                        c = pl.ds(q * LANES, LANES)
                        acc = nrows_v[b, base, c]
                        for k in range(1, K):
                            acc = acc + nrows_v[b, base + k, c]
                        out_v[b, r, c] = acc * (1.0 / K)
                    return 0

                lax.fori_loop(0, CHUNK, row_body, 0, unroll=2)
                pltpu.async_copy(
                    out_v.at[b],
                    out_hbm.at[pl.ds(row0 + g * CHUNK, CHUNK), pl.ds(0, D)],
                    osem[b])

                @pl.when(g + NBUF < n_chunks)
                def _():
                    fire(g + NBUF, b)

        for b in range(NBUF):
            wait_out(b)

    return agg


def _make_igather(B: int):
    info = plsc.get_sparse_core_info()
    NC, NS = info.num_cores, info.num_subcores
    NW = NC * NS
    b_per_w = B // NW

    mesh = plsc.VectorSubcoreMesh(core_axis_name="c", subcore_axis_name="s")

    @functools.partial(
        pl.kernel,
        mesh=mesh,
        compiler_params=pltpu.CompilerParams(use_tc_tiling_on_sc=False),
        out_type=jax.ShapeDtypeStruct((B, 2 * D), jnp.float32),
        scratch_types=[
            pltpu.VMEM((b_per_w,), jnp.int32),
            pltpu.VMEM((b_per_w, D), jnp.float32),
            pltpu.SemaphoreType.DMA,
        ],
    )
    def igather(item_idx_hbm, item_tab_hbm, out_hbm, iidx_v, irows_v, sem):
        wid = lax.axis_index("s") * NC + lax.axis_index("c")
        row0 = wid * b_per_w
        pltpu.sync_copy(item_idx_hbm.at[pl.ds(row0, b_per_w)], iidx_v)
        pltpu.async_copy(item_tab_hbm.at[iidx_v], irows_v, sem).wait()
        pltpu.sync_copy(irows_v,
                        out_hbm.at[pl.ds(row0, b_per_w), pl.ds(0, D)])

    return igather


TBLK = 16384
TBLK_BITS = 14


def _tr_body(a_ref, b_ref, o_ref):
    o_ref[...] = jnp.concatenate([a_ref[...].T, b_ref[...].T], axis=1)


def _make_tr(R: int):
    npairs = (R + 2 * TBLK - 1) // (2 * TBLK)
    nblocks = (R + TBLK - 1) // TBLK
    return pl.pallas_call(
        _tr_body,
        grid=(npairs,),
        in_specs=[
            pl.BlockSpec((D, TBLK), lambda j: (0, 2 * j)),
            pl.BlockSpec((D, TBLK),
                         lambda j: (0, jnp.minimum(2 * j + 1, nblocks - 1))),
        ],
        out_specs=pl.BlockSpec((TBLK, 2 * D), lambda j: (j, 0)),
        out_shape=jax.ShapeDtypeStruct((npairs * TBLK, 2 * D), jnp.float32),
    )


def _remap(i):
    blk = i >> TBLK_BITS
    return ((((blk >> 1) << TBLK_BITS) + (i & (TBLK - 1))) << 1) + (blk & 1)


def _fc_body(a_ref, i_ref, w_ref, b_ref, o_ref):
    x = a_ref[:, 0:D] + i_ref[:, 0:D]
    y = lax.dot_general(x, w_ref[...], (((1,), (1,)), ((), ())),
                        preferred_element_type=jnp.float32)
    o_ref[...] = jnp.maximum(y + b_ref[...], 0.0)


def _make_fc(B: int):
    blk = 2048
    return pl.pallas_call(
        _fc_body,
        grid=(B // blk,),
        in_specs=[
            pl.BlockSpec((blk, 2 * D), lambda i: (i, 0)),
            pl.BlockSpec((blk, 2 * D), lambda i: (i, 0)),
            pl.BlockSpec((D, D), lambda i: (0, 0)),
            pl.BlockSpec((1, D), lambda i: (0, 0)),
        ],
        out_specs=pl.BlockSpec((blk, D), lambda i: (i, 0)),
        out_shape=jax.ShapeDtypeStruct((B, D), jnp.float32),
    )


def kernel(item_indices, neighbor_indices, item_table, entity_table, fc1_w, fc1_b):
    B = item_indices.shape[0]
    NE = entity_table.shape[0]
    NI = item_table.shape[0]
    ent_t = entity_table.T
    itm_t = item_table.T
    ent_lin = _make_tr(NE)(ent_t, ent_t)
    ent_lin = ent_lin.reshape(2 * ent_lin.shape[0], D)
    itm_lin = _make_tr(NI)(itm_t, itm_t)
    itm_lin = itm_lin.reshape(2 * itm_lin.shape[0], D)
    nb = _remap(neighbor_indices.reshape(-1).astype(jnp.int32))
    ii = _remap(item_indices.astype(jnp.int32))
    agg = _make_agg(B)(nb, ent_lin)
    item_emb = _make_igather(B)(ii, itm_lin)
    return _make_fc(B)(agg, item_emb, fc1_w, fc1_b.reshape(1, D))


# speedup vs baseline: 1.2161x; 1.1030x over previous
import functools

import jax
import jax.numpy as jnp
from jax import lax
from jax.experimental import pallas as pl
from jax.experimental.pallas import tpu as pltpu
from jax.experimental.pallas import tpu_sc as plsc

D = 64
K = 50
LANES = 16
CHUNK = 8
NBUF = 2
CHUNK_IDX = CHUNK * K
IDX_SLICES = [(0, CHUNK_IDX)]


def _make_agg(B: int):
    info = plsc.get_sparse_core_info()
    NC, NS = info.num_cores, info.num_subcores
    NW = NC * NS
    assert B % (NW * CHUNK * NBUF) == 0
    b_per_w = B // NW
    n_chunks = b_per_w // CHUNK

    mesh = plsc.VectorSubcoreMesh(core_axis_name="c", subcore_axis_name="s")

    @functools.partial(
        pl.kernel,
        mesh=mesh,
        compiler_params=pltpu.CompilerParams(use_tc_tiling_on_sc=False),
        out_type=jax.ShapeDtypeStruct((B, 2 * D), jnp.float32),
        scratch_types=[
            pltpu.VMEM((b_per_w * K,), jnp.int32),
            pltpu.VMEM((NBUF, CHUNK_IDX, D), jnp.float32),
            pltpu.VMEM((NBUF, CHUNK, D), jnp.float32),
            [pltpu.SemaphoreType.DMA] * NBUF,
            [pltpu.SemaphoreType.DMA] * NBUF,
        ],
    )
    def agg(nbr_idx_hbm, ent_tab_hbm, out_hbm, nidx_v, nrows_v, out_v,
            gsem, osem):
        wid = lax.axis_index("s") * NC + lax.axis_index("c")
        row0 = wid * b_per_w
        pltpu.sync_copy(nbr_idx_hbm.at[pl.ds(row0 * K, b_per_w * K)], nidx_v)

        def fire(g, b):
            pltpu.async_copy(
                ent_tab_hbm.at[nidx_v.at[pl.ds(g * CHUNK_IDX, CHUNK_IDX)]],
                nrows_v.at[b], gsem[b])

        def wait_gather(b):
            pltpu.make_async_copy(
                ent_tab_hbm.at[pl.ds(0, CHUNK_IDX)], nrows_v.at[b],
                gsem[b]).wait()

        def wait_out(b):
            pltpu.make_async_copy(
                out_v.at[b], out_hbm.at[pl.ds(0, CHUNK), pl.ds(0, D)],
                osem[b]).wait()

        for b in range(NBUF):
            fire(b, b)

        @pl.loop(0, n_chunks, step=NBUF)
        def outer(i):
            for b in range(NBUF):
                g = i + b
                wait_gather(b)

                @pl.when(g >= NBUF)
                def _():
                    wait_out(b)

                def row_body(r, _):
                    base = r * K
                    for q in range(D // LANES):
                        c = pl.ds(q * LANES, LANES)
                        acc = nrows_v[b, base, c]
                        for k in range(1, K):
                            acc = acc + nrows_v[b, base + k, c]
                        out_v[b, r, c] = acc * (1.0 / K)
                    return 0

                lax.fori_loop(0, CHUNK, row_body, 0, unroll=2)
                pltpu.async_copy(
                    out_v.at[b],
                    out_hbm.at[pl.ds(row0 + g * CHUNK, CHUNK), pl.ds(0, D)],
                    osem[b])

                @pl.when(g + NBUF < n_chunks)
                def _():
                    fire(g + NBUF, b)

        for b in range(NBUF):
            wait_out(b)

    return agg


def _make_igather(B: int):
    info = plsc.get_sparse_core_info()
    NC, NS = info.num_cores, info.num_subcores
    NW = NC * NS
    b_per_w = B // NW

    mesh = plsc.VectorSubcoreMesh(core_axis_name="c", subcore_axis_name="s")

    @functools.partial(
        pl.kernel,
        mesh=mesh,
        compiler_params=pltpu.CompilerParams(use_tc_tiling_on_sc=False),
        out_type=jax.ShapeDtypeStruct((B, 2 * D), jnp.float32),
        scratch_types=[
            pltpu.VMEM((b_per_w,), jnp.int32),
            pltpu.VMEM((b_per_w, D), jnp.float32),
            pltpu.SemaphoreType.DMA,
        ],
    )
    def igather(item_idx_hbm, item_tab_hbm, out_hbm, iidx_v, irows_v, sem):
        wid = lax.axis_index("s") * NC + lax.axis_index("c")
        row0 = wid * b_per_w
        pltpu.sync_copy(item_idx_hbm.at[pl.ds(row0, b_per_w)], iidx_v)
        pltpu.async_copy(item_tab_hbm.at[iidx_v], irows_v, sem).wait()
        pltpu.sync_copy(irows_v,
                        out_hbm.at[pl.ds(row0, b_per_w), pl.ds(0, D)])

    return igather


TBLK = 16384
TBLK_BITS = 14


def _tr_body(x_ref, o_ref):
    x = x_ref[...]
    y = jnp.concatenate([x[:, :TBLK // 2], x[:, TBLK // 2:]], axis=0)
    o_ref[...] = y.T


def _make_tr(R: int):
    nblocks = (R + TBLK - 1) // TBLK
    return pl.pallas_call(
        _tr_body,
        grid=(nblocks,),
        in_specs=[pl.BlockSpec((D, TBLK), lambda j: (0, j))],
        out_specs=pl.BlockSpec((TBLK // 2, 2 * D), lambda j: (j, 0)),
        out_shape=jax.ShapeDtypeStruct((nblocks * TBLK // 2, 2 * D),
                                       jnp.float32),
    )


def _remap(i):
    return (((i >> TBLK_BITS) << TBLK_BITS)
            + ((i & (TBLK // 2 - 1)) << 1)
            + ((i >> (TBLK_BITS - 1)) & 1))


def _fc_body(a_ref, i_ref, w_ref, b_ref, o_ref):
    x = a_ref[:, 0:D] + i_ref[:, 0:D]
    y = lax.dot_general(x, w_ref[...], (((1,), (1,)), ((), ())),
                        preferred_element_type=jnp.float32)
    o_ref[...] = jnp.maximum(y + b_ref[...], 0.0)


def _make_fc(B: int):
    blk = 2048
    return pl.pallas_call(
        _fc_body,
        grid=(B // blk,),
        in_specs=[
            pl.BlockSpec((blk, 2 * D), lambda i: (i, 0)),
            pl.BlockSpec((blk, 2 * D), lambda i: (i, 0)),
            pl.BlockSpec((D, D), lambda i: (0, 0)),
            pl.BlockSpec((1, D), lambda i: (0, 0)),
        ],
        out_specs=pl.BlockSpec((blk, D), lambda i: (i, 0)),
        out_shape=jax.ShapeDtypeStruct((B, D), jnp.float32),
    )


def kernel(item_indices, neighbor_indices, item_table, entity_table, fc1_w, fc1_b):
    B = item_indices.shape[0]
    NE = entity_table.shape[0]
    NI = item_table.shape[0]
    ent_t = entity_table.T
    itm_t = item_table.T
    ent_lin = _make_tr(NE)(ent_t)
    ent_lin = ent_lin.reshape(2 * ent_lin.shape[0], D)
    itm_lin = _make_tr(NI)(itm_t)
    itm_lin = itm_lin.reshape(2 * itm_lin.shape[0], D)
    nb = _remap(neighbor_indices.reshape(-1).astype(jnp.int32))
    ii = _remap(item_indices.astype(jnp.int32))
    agg = _make_agg(B)(nb, ent_lin)
    item_emb = _make_igather(B)(ii, itm_lin)
    return _make_fc(B)(agg, item_emb, fc1_w, fc1_b.reshape(1, D))


# speedup vs baseline: 1.2411x; 1.0205x over previous
import functools

import jax
import jax.numpy as jnp
from jax import lax
from jax.experimental import pallas as pl
from jax.experimental.pallas import tpu as pltpu
from jax.experimental.pallas import tpu_sc as plsc

D = 64
K = 50
LANES = 16
CHUNK = 8
NBUF = 2
CHUNK_IDX = CHUNK * K
IDX_SLICES = [(0, CHUNK_IDX)]


def _make_agg(B: int):
    info = plsc.get_sparse_core_info()
    NC, NS = info.num_cores, info.num_subcores
    NW = NC * NS
    assert B % (NW * CHUNK * NBUF) == 0
    b_per_w = B // NW
    n_chunks = b_per_w // CHUNK

    mesh = plsc.VectorSubcoreMesh(core_axis_name="c", subcore_axis_name="s")

    @functools.partial(
        pl.kernel,
        mesh=mesh,
        compiler_params=pltpu.CompilerParams(use_tc_tiling_on_sc=False),
        out_type=jax.ShapeDtypeStruct((B, 2 * D), jnp.float32),
        scratch_types=[
            pltpu.VMEM((b_per_w * K,), jnp.int32),
            pltpu.VMEM((NBUF, CHUNK_IDX, D), jnp.float32),
            pltpu.VMEM((NBUF, CHUNK, D), jnp.float32),
            [pltpu.SemaphoreType.DMA] * NBUF,
            [pltpu.SemaphoreType.DMA] * NBUF,
        ],
    )
    def agg(nbr_idx_hbm, ent_tab_hbm, out_hbm, nidx_v, nrows_v, out_v,
            gsem, osem):
        wid = lax.axis_index("s") * NC + lax.axis_index("c")
        row0 = wid * b_per_w
        pltpu.sync_copy(nbr_idx_hbm.at[pl.ds(row0 * K, b_per_w * K)], nidx_v)

        def fire(g, b):
            pltpu.async_copy(
                ent_tab_hbm.at[nidx_v.at[pl.ds(g * CHUNK_IDX, CHUNK_IDX)]],
                nrows_v.at[b], gsem[b])

        def wait_gather(b):
            pltpu.make_async_copy(
                ent_tab_hbm.at[pl.ds(0, CHUNK_IDX)], nrows_v.at[b],
                gsem[b]).wait()

        def wait_out(b):
            pltpu.make_async_copy(
                out_v.at[b], out_hbm.at[pl.ds(0, CHUNK), pl.ds(0, D)],
                osem[b]).wait()

        for b in range(NBUF):
            fire(b, b)

        @pl.loop(0, n_chunks, step=NBUF)
        def outer(i):
            for b in range(NBUF):
                g = i + b
                wait_gather(b)

                @pl.when(g >= NBUF)
                def _():
                    wait_out(b)

                def row_body(r, _):
                    base = r * K
                    for q in range(D // LANES):
                        c = pl.ds(q * LANES, LANES)
                        acc = nrows_v[b, base, c]
                        for k in range(1, K):
                            acc = acc + nrows_v[b, base + k, c]
                        out_v[b, r, c] = acc * (1.0 / K)
                    return 0

                lax.fori_loop(0, CHUNK, row_body, 0, unroll=2)
                pltpu.async_copy(
                    out_v.at[b],
                    out_hbm.at[pl.ds(row0 + g * CHUNK, CHUNK), pl.ds(0, D)],
                    osem[b])

                @pl.when(g + NBUF < n_chunks)
                def _():
                    fire(g + NBUF, b)

        for b in range(NBUF):
            wait_out(b)

    return agg


def _make_igather(B: int):
    info = plsc.get_sparse_core_info()
    NC, NS = info.num_cores, info.num_subcores
    NW = NC * NS
    b_per_w = B // NW

    mesh = plsc.VectorSubcoreMesh(core_axis_name="c", subcore_axis_name="s")

    @functools.partial(
        pl.kernel,
        mesh=mesh,
        compiler_params=pltpu.CompilerParams(use_tc_tiling_on_sc=False),
        out_type=jax.ShapeDtypeStruct((B, 2 * D), jnp.float32),
        scratch_types=[
            pltpu.VMEM((b_per_w,), jnp.int32),
            pltpu.VMEM((b_per_w, D), jnp.float32),
            pltpu.SemaphoreType.DMA,
        ],
    )
    def igather(item_idx_hbm, item_tab_hbm, out_hbm, iidx_v, irows_v, sem):
        wid = lax.axis_index("s") * NC + lax.axis_index("c")
        row0 = wid * b_per_w
        pltpu.sync_copy(item_idx_hbm.at[pl.ds(row0, b_per_w)], iidx_v)
        pltpu.async_copy(item_tab_hbm.at[iidx_v], irows_v, sem).wait()
        pltpu.sync_copy(irows_v,
                        out_hbm.at[pl.ds(row0, b_per_w), pl.ds(0, D)])

    return igather


TBLK = 16384
TBLK_BITS = 14


def _tr_body(x_ref, o_ref):
    x = x_ref[...]
    y = jnp.concatenate([x[:, :TBLK // 2], x[:, TBLK // 2:]], axis=0)
    o_ref[...] = y.T


def _make_tr(R: int):
    nblocks = (R + TBLK - 1) // TBLK
    return pl.pallas_call(
        _tr_body,
        grid=(nblocks,),
        in_specs=[pl.BlockSpec((D, TBLK), lambda j: (0, j))],
        out_specs=pl.BlockSpec((TBLK // 2, 2 * D), lambda j: (j, 0)),
        out_shape=jax.ShapeDtypeStruct((nblocks * TBLK // 2, 2 * D),
                                       jnp.float32),
    )


def _remap(i):
    return (((i >> TBLK_BITS) << TBLK_BITS)
            + ((i & (TBLK // 2 - 1)) << 1)
            + ((i >> (TBLK_BITS - 1)) & 1))


def _fc_body(a_ref, i_ref, w_ref, b_ref, o_ref):
    x = a_ref[:, 0:D] + i_ref[:, 0:D]
    y = lax.dot_general(x, w_ref[...], (((1,), (1,)), ((), ())),
                        preferred_element_type=jnp.float32)
    o_ref[...] = jnp.maximum(y + b_ref[...], 0.0).T


def _make_fc(B: int):
    blk = 2048
    return pl.pallas_call(
        _fc_body,
        grid=(B // blk,),
        in_specs=[
            pl.BlockSpec((blk, 2 * D), lambda i: (i, 0)),
            pl.BlockSpec((blk, 2 * D), lambda i: (i, 0)),
            pl.BlockSpec((D, D), lambda i: (0, 0)),
            pl.BlockSpec((1, D), lambda i: (0, 0)),
        ],
        out_specs=pl.BlockSpec((D, blk), lambda i: (0, i)),
        out_shape=jax.ShapeDtypeStruct((D, B), jnp.float32),
    )


def kernel(item_indices, neighbor_indices, item_table, entity_table, fc1_w, fc1_b):
    B = item_indices.shape[0]
    NE = entity_table.shape[0]
    NI = item_table.shape[0]
    ent_t = entity_table.T
    itm_t = item_table.T
    ent_lin = _make_tr(NE)(ent_t)
    ent_lin = ent_lin.reshape(2 * ent_lin.shape[0], D)
    itm_lin = _make_tr(NI)(itm_t)
    itm_lin = itm_lin.reshape(2 * itm_lin.shape[0], D)
    nb = _remap(neighbor_indices.reshape(-1).astype(jnp.int32))
    ii = _remap(item_indices.astype(jnp.int32))
    agg = _make_agg(B)(nb, ent_lin)
    item_emb = _make_igather(B)(ii, itm_lin)
    return _make_fc(B)(agg, item_emb, fc1_w, fc1_b.reshape(1, D)).T
